# Initial kernel scaffold; baseline (speedup 1.0000x reference)
import jax
import jax.numpy as jnp
from jax.experimental import pallas as pl


def kernel(x, edge_index, edge_attr, params):
    raise NotImplementedError("write your pallas kernel here")


# speedup vs baseline: 4.3161x; 4.3161x over previous
import functools

import jax
import jax.numpy as jnp
from jax import lax
from jax.experimental import pallas as pl
from jax.experimental.pallas import tpu as pltpu
from jax.experimental.pallas import tpu_sc as plsc

_NC = 2
_NS = 16
_LN = 16
_W = 40
_SW = 50


def _mlp_pallas(x_list, layers, relu_flags, block_rows):
    rows, din = x_list[0].shape
    n_in = len(x_list)
    n_lay = len(layers)
    dout = layers[-1][0].shape[1]

    def body(*refs):
        x_refs = refs[:n_in]
        w_refs = refs[n_in:n_in + 2 * n_lay]
        o_ref = refs[-1]
        acc = x_refs[0][...]
        for r in x_refs[1:]:
            acc = acc + r[...]
        for li in range(n_lay):
            w = w_refs[2 * li][...]
            b = w_refs[2 * li + 1][...]
            acc = jnp.dot(acc, w, preferred_element_type=jnp.float32) + b
            if relu_flags[li]:
                acc = jnp.maximum(acc, 0.0)
        o_ref[...] = acc

    in_specs = [
        pl.BlockSpec((block_rows, din), lambda i: (i, 0)) for _ in range(n_in)
    ]
    flat = []
    for w, b in layers:
        in_specs.append(pl.BlockSpec(w.shape, lambda i: (0, 0)))
        in_specs.append(pl.BlockSpec((1, b.shape[0]), lambda i: (0, 0)))
        flat += [w, b.reshape(1, -1)]

    return pl.pallas_call(
        body,
        grid=(rows // block_rows,),
        in_specs=in_specs,
        out_specs=pl.BlockSpec((block_rows, dout), lambda i: (i, 0)),
        out_shape=jax.ShapeDtypeStruct((rows, dout), jnp.float32),
    )(*x_list, *flat)


def _mlp_layers(p):
    return [(p["W1"], p["b1"]), (p["W2"], p["b2"]), (p["W3"], p["b3"])]


def _sc_message(h, e, src5, dst5):
    n, d = h.shape
    nsw = src5.shape[2]
    epw = nsw * _SW * _W
    npad = -(-n // (_NS * 8)) * (_NS * 8)
    rps = npad // _NS
    nfull = rps // _W
    tail = rps - nfull * _W
    mesh = plsc.VectorSubcoreMesh(core_axis_name="c", subcore_axis_name="s")

    @functools.partial(
        pl.kernel,
        out_type=jax.ShapeDtypeStruct((_NC, npad, d), jnp.float32),
        mesh=mesh,
        scratch_types=[
            pltpu.VMEM((_SW, _W), jnp.int32),
            pltpu.VMEM((_SW, _W), jnp.int32),
            pltpu.VMEM((_W, d), jnp.float32),
            pltpu.VMEM((_W, d), jnp.float32),
            pltpu.VMEM((_W, d), jnp.float32),
            pltpu.VMEM((_W, d), jnp.float32),
            pltpu.VMEM_SHARED((npad, d), jnp.float32),
            pltpu.SemaphoreType.DMA,
            pltpu.SemaphoreType.DMA,
            pltpu.SemaphoreType.DMA,
            pltpu.SemaphoreType.DMA,
        ],
    )
    def k(h_hbm, e_hbm, src_hbm, dst_hbm, out_hbm,
          src_v, dst_v, h_a, e_a, h_b, e_b, agg,
          sem_ha, sem_ea, sem_hb, sem_eb):
        c = lax.axis_index("c")
        s = lax.axis_index("s")
        wbase = (c * _NS + s) * epw

        @pl.loop(0, _W)
        def _(i):
            for j in range(d // _LN):
                h_a[i, pl.ds(j * _LN, _LN)] = jnp.zeros((_LN,), jnp.float32)

        @pl.loop(0, nfull)
        def _(q):
            pltpu.sync_copy(h_a, agg.at[pl.ds(s * rps + q * _W, _W), :])
        if tail:
            pltpu.sync_copy(
                h_a.at[pl.ds(0, tail), :],
                agg.at[pl.ds(s * rps + nfull * _W, tail), :])

        plsc.subcore_barrier()

        def fire(sw, g, hbuf, ebuf, sem_h, sem_e):
            pltpu.async_copy(h_hbm.at[src_v.at[g]], hbuf, sem_h)
            pltpu.async_copy(
                e_hbm.at[pl.ds(wbase + (sw * _SW + g) * _W, _W), :],
                ebuf, sem_e)

        def drain(hbuf, ebuf, sem_h, sem_e):
            pltpu.make_async_copy(h_hbm.at[pl.ds(0, _W), :], hbuf, sem_h).wait()
            pltpu.make_async_copy(e_hbm.at[pl.ds(0, _W), :], ebuf, sem_e).wait()

        def compute_scatter(g, hbuf, ebuf):
            @pl.loop(0, _W)
            def _(i):
                for j in range(d // _LN):
                    sl = pl.ds(j * _LN, _LN)
                    hbuf[i, sl] = jnp.maximum(hbuf[i, sl] + ebuf[i, sl], 0.0)
            pltpu.sync_copy(hbuf, agg.at[dst_v.at[g]], add=True)

        @pl.loop(0, nsw)
        def _(sw):
            pltpu.sync_copy(src_hbm.at[c, s, sw], src_v)
            pltpu.sync_copy(dst_hbm.at[c, s, sw], dst_v)
            fire(sw, 0, h_a, e_a, sem_ha, sem_ea)
            fire(sw, 1, h_b, e_b, sem_hb, sem_eb)

            @pl.loop(0, _SW, step=2)
            def _(g):
                drain(h_a, e_a, sem_ha, sem_ea)
                compute_scatter(g, h_a, e_a)

                @pl.when(g + 2 < _SW)
                def _():
                    fire(sw, g + 2, h_a, e_a, sem_ha, sem_ea)

                drain(h_b, e_b, sem_hb, sem_eb)
                compute_scatter(g + 1, h_b, e_b)

                @pl.when(g + 3 < _SW)
                def _():
                    fire(sw, g + 3, h_b, e_b, sem_hb, sem_eb)

        plsc.subcore_barrier()

        @pl.loop(0, nfull)
        def _(q):
            base = s * rps + q * _W
            pltpu.sync_copy(agg.at[pl.ds(base, _W), :],
                            out_hbm.at[c, pl.ds(base, _W), :])
        if tail:
            base = s * rps + nfull * _W
            pltpu.sync_copy(agg.at[pl.ds(base, tail), :],
                            out_hbm.at[c, pl.ds(base, tail), :])

    return k(h, e, src5, dst5)[:, :n, :]


def kernel(x, edge_index, edge_attr, params):
    n = x.shape[0]
    e_cnt = edge_attr.shape[0]
    nsw = e_cnt // (_NC * _NS * _SW * _W)
    src5 = edge_index[0].astype(jnp.int32).reshape(_NC, _NS, nsw, _SW, _W)
    dst5 = edge_index[1].astype(jnp.int32).reshape(_NC, _NS, nsw, _SW, _W)

    h = _mlp_pallas([x], _mlp_layers(params["node_to_node"]),
                    (True, True, False), block_rows=2000)
    e = _mlp_pallas([edge_attr], _mlp_layers(params["edge_to_node"]),
                    (True, True, False), block_rows=2000)

    parts = _sc_message(h, e, src5, dst5)
    h = _mlp_pallas([h, parts[0], parts[1]], _mlp_layers(params["gine"][0]),
                    (True, True, False), block_rows=2000)

    parts = _sc_message(h, e, src5, dst5)
    return _mlp_pallas(
        [h, parts[0], parts[1]],
        _mlp_layers(params["gine"][1]) + _mlp_layers(params["final_mlp"]),
        (True, True, False, True, True, False), block_rows=2000)
